# baseline (speedup 1.0000x reference)
import functools

import jax
import jax.numpy as jnp
import numpy as np
from jax import lax
from jax.experimental import pallas as pl
from jax.experimental.pallas import tpu as pltpu
from jax.experimental.pallas import tpu_sc as plsc

N = 10000
D = 128
E = 320000
NC = 2
NS = 16
EW = E // (NC * NS)
CH = 25
NBUF = 4
NSUP = EW // (2 * NBUF * CH)
NP = 10240
RPT = NP // NS
ZR = 16
BN_EPS = 1e-5


def _sc_aggregate(h, src3, dst3):
    mesh = plsc.VectorSubcoreMesh(core_axis_name="c", subcore_axis_name="s")

    @functools.partial(
        pl.kernel,
        out_type=jax.ShapeDtypeStruct((NC, NP, D), jnp.float32),
        mesh=mesh,
        scratch_types=[
            pltpu.VMEM((2, 2 * NBUF, CH), jnp.int32),
            pltpu.VMEM((2, 2 * NBUF, CH), jnp.int32),
            *[pltpu.VMEM((CH, D), jnp.float32) for _ in range(2 * NBUF)],
            pltpu.VMEM((ZR, D), jnp.float32),
            pltpu.VMEM_SHARED((NP, D), jnp.float32),
            *[pltpu.SemaphoreType.DMA for _ in range(4 * NBUF + 2)],
        ],
    )
    def agg_kernel(h_hbm, src_hbm, dst_hbm, out_hbm, sbuf, dbuf, *rest):
        rows = rest[:2 * NBUF]
        zbuf = rest[2 * NBUF]
        agg = rest[2 * NBUF + 1]
        gsem = rest[2 * NBUF + 2:2 * NBUF + 2 + 2 * NBUF]
        ssem = rest[2 * NBUF + 2 + 2 * NBUF:2 * NBUF + 2 + 4 * NBUF]
        isem = rest[2 * NBUF + 2 + 4 * NBUF:]
        cid = lax.axis_index("c")
        sid = lax.axis_index("s")

        zvec = jnp.zeros((16,), jnp.float32)

        def zfill(k, carry):
            zbuf[k // (D // 16), pl.ds((k % (D // 16)) * 16, 16)] = zvec
            return carry

        lax.fori_loop(0, ZR * (D // 16), zfill, 0)
        for r in range(RPT // ZR):
            pltpu.sync_copy(zbuf, agg.at[pl.ds(sid * RPT + r * ZR, ZR)])
        plsc.subcore_barrier()

        pltpu.sync_copy(src_hbm.at[cid, sid, 0], sbuf.at[0])
        pltpu.sync_copy(dst_hbm.at[cid, sid, 0], dbuf.at[0])
        for j in range(2 * NBUF):
            pltpu.async_copy(h_hbm.at[sbuf.at[0, j]], rows[j], gsem[j])

        def super_step(s, carry):
            par = s % 2
            nxt = (s + 1) % 2

            @pl.when(s + 1 < NSUP)
            def _():
                pltpu.async_copy(src_hbm.at[cid, sid, s + 1], sbuf.at[nxt],
                                 isem[0])
                pltpu.async_copy(dst_hbm.at[cid, sid, s + 1], dbuf.at[nxt],
                                 isem[1])

            scat_a = []
            for b in range(NBUF):
                pltpu.make_async_copy(
                    h_hbm.at[sbuf.at[par, b]], rows[b], gsem[b]).wait()
                scat_a.append(pltpu.async_copy(
                    rows[b], agg.at[dbuf.at[par, b]], ssem[b], add=True))

            @pl.when(s + 1 < NSUP)
            def _():
                pltpu.make_async_copy(src_hbm.at[cid, sid, s + 1],
                                      sbuf.at[nxt], isem[0]).wait()
                pltpu.make_async_copy(dst_hbm.at[cid, sid, s + 1],
                                      dbuf.at[nxt], isem[1]).wait()
                for b in range(NBUF):
                    scat_a[b].wait()
                    pltpu.async_copy(h_hbm.at[sbuf.at[nxt, b]], rows[b],
                                     gsem[b])

            scat_b = []
            for b in range(NBUF):
                j = NBUF + b
                pltpu.make_async_copy(
                    h_hbm.at[sbuf.at[par, j]], rows[j], gsem[j]).wait()
                scat_b.append(pltpu.async_copy(
                    rows[j], agg.at[dbuf.at[par, j]], ssem[j], add=True))

            @pl.when(s + 1 < NSUP)
            def _():
                for b in range(NBUF):
                    j = NBUF + b
                    scat_b[b].wait()
                    pltpu.async_copy(h_hbm.at[sbuf.at[nxt, j]], rows[j],
                                     gsem[j])

            return carry

        lax.fori_loop(0, NSUP, super_step, 0)
        for j in range(2 * NBUF):
            pltpu.make_async_copy(
                rows[j], agg.at[dbuf.at[(NSUP - 1) % 2, j]], ssem[j]).wait()

        plsc.subcore_barrier()
        pltpu.sync_copy(agg.at[pl.ds(sid * RPT, RPT)],
                        out_hbm.at[cid, pl.ds(sid * RPT, RPT)])

    return agg_kernel(h, src3, dst3)


def _tc_mlp(h, p, eps, W1, b1, g, bt, W2, b2, relu_out):
    BR = 2000
    inv = float(1.0 / np.sqrt(1.0 + BN_EPS))

    def body(eps_ref, h_ref, p0_ref, p1_ref, w1_ref, b1_ref, g_ref, bt_ref,
             w2_ref, b2_ref, o_ref):
        scale = 1.0 + eps_ref[0, 0]
        a = h_ref[...] * scale + p0_ref[0] + p1_ref[0]
        gs = g_ref[...] * inv
        w1 = w1_ref[...] * gs
        c1 = b1_ref[...] * gs + bt_ref[...]
        t = jnp.dot(a, w1, preferred_element_type=jnp.float32,
                    precision=jax.lax.Precision.HIGHEST) + c1
        t = jnp.maximum(t, 0.0)
        o = jnp.dot(t, w2_ref[...], preferred_element_type=jnp.float32,
                    precision=jax.lax.Precision.HIGHEST) + b2_ref[...]
        if relu_out:
            o = jnp.maximum(o, 0.0)
        o_ref[...] = o

    row_spec = pl.BlockSpec((BR, D), lambda i: (i, 0))
    p0_spec = pl.BlockSpec((1, BR, D), lambda i: (0, i, 0))
    p1_spec = pl.BlockSpec((1, BR, D), lambda i: (1, i, 0))
    full = lambda shape: pl.BlockSpec(shape, lambda i: (0,) * len(shape))
    return pl.pallas_call(
        body,
        grid=(N // BR,),
        in_specs=[
            pl.BlockSpec(memory_space=pltpu.SMEM),
            row_spec, p0_spec, p1_spec,
            full((D, D)), full((1, D)), full((1, D)), full((1, D)),
            full((D, D)), full((1, D)),
        ],
        out_specs=row_spec,
        out_shape=jax.ShapeDtypeStruct((N, D), jnp.float32),
    )(eps.reshape(1, 1), h, p, p, W1, b1.reshape(1, D), g.reshape(1, D),
      bt.reshape(1, D), W2, b2.reshape(1, D))


def kernel(x, edge_index,
           W1_0, b1_0, g_0, bt_0, W2_0, b2_0, eps_0,
           W1_1, b1_1, g_1, bt_1, W2_1, b2_1, eps_1,
           W1_2, b1_2, g_2, bt_2, W2_2, b2_2, eps_2):
    src3 = edge_index[0].reshape(NC, NS, NSUP, 2 * NBUF, CH)
    dst3 = edge_index[1].reshape(NC, NS, NSUP, 2 * NBUF, CH)
    layers = [
        (W1_0, b1_0, g_0, bt_0, W2_0, b2_0, eps_0),
        (W1_1, b1_1, g_1, bt_1, W2_1, b2_1, eps_1),
        (W1_2, b1_2, g_2, bt_2, W2_2, b2_2, eps_2),
    ]
    h = x
    for i, (W1, b1, g, bt, W2, b2, eps) in enumerate(layers):
        p = _sc_aggregate(h, src3, dst3)
        h = _tc_mlp(h, p, eps, W1, b1, g, bt, W2, b2, relu_out=(i < 2))
    return h


# speedup vs baseline: 1.1080x; 1.1080x over previous
import functools

import jax
import jax.numpy as jnp
import numpy as np
from jax import lax
from jax.experimental import pallas as pl
from jax.experimental.pallas import tpu as pltpu
from jax.experimental.pallas import tpu_sc as plsc

N = 10000
D = 128
E = 320000
NC = 2
NS = 16
EW = E // (NC * NS)
CH = 50
NBUF = 2
NSUP = EW // (2 * NBUF * CH)
NP = 10240
RPT = NP // NS
ZR = 16
BN_EPS = 1e-5


def _sc_aggregate(h, src3, dst3):
    mesh = plsc.VectorSubcoreMesh(core_axis_name="c", subcore_axis_name="s")

    @functools.partial(
        pl.kernel,
        out_type=jax.ShapeDtypeStruct((NC, NP, D), jnp.float32),
        mesh=mesh,
        scratch_types=[
            pltpu.VMEM((2, 2 * NBUF, CH), jnp.int32),
            pltpu.VMEM((2, 2 * NBUF, CH), jnp.int32),
            *[pltpu.VMEM((CH, D), jnp.float32) for _ in range(2 * NBUF)],
            pltpu.VMEM((ZR, D), jnp.float32),
            pltpu.VMEM_SHARED((NP, D), jnp.float32),
            *[pltpu.SemaphoreType.DMA for _ in range(4 * NBUF + 2)],
        ],
    )
    def agg_kernel(h_hbm, src_hbm, dst_hbm, out_hbm, sbuf, dbuf, *rest):
        rows = rest[:2 * NBUF]
        zbuf = rest[2 * NBUF]
        agg = rest[2 * NBUF + 1]
        gsem = rest[2 * NBUF + 2:2 * NBUF + 2 + 2 * NBUF]
        ssem = rest[2 * NBUF + 2 + 2 * NBUF:2 * NBUF + 2 + 4 * NBUF]
        isem = rest[2 * NBUF + 2 + 4 * NBUF:]
        cid = lax.axis_index("c")
        sid = lax.axis_index("s")

        zvec = jnp.zeros((16,), jnp.float32)

        def zfill(k, carry):
            zbuf[k // (D // 16), pl.ds((k % (D // 16)) * 16, 16)] = zvec
            return carry

        lax.fori_loop(0, ZR * (D // 16), zfill, 0)
        for r in range(RPT // ZR):
            pltpu.sync_copy(zbuf, agg.at[pl.ds(sid * RPT + r * ZR, ZR)])
        plsc.subcore_barrier()

        pltpu.sync_copy(src_hbm.at[cid, sid, 0], sbuf.at[0])
        pltpu.sync_copy(dst_hbm.at[cid, sid, 0], dbuf.at[0])
        for j in range(2 * NBUF):
            pltpu.async_copy(h_hbm.at[sbuf.at[0, j]], rows[j], gsem[j])

        def super_step(s, carry):
            par = s % 2
            nxt = (s + 1) % 2

            @pl.when(s + 1 < NSUP)
            def _():
                pltpu.async_copy(src_hbm.at[cid, sid, s + 1], sbuf.at[nxt],
                                 isem[0])
                pltpu.async_copy(dst_hbm.at[cid, sid, s + 1], dbuf.at[nxt],
                                 isem[1])

            scat_a = []
            for b in range(NBUF):
                pltpu.make_async_copy(
                    h_hbm.at[sbuf.at[par, b]], rows[b], gsem[b]).wait()
                scat_a.append(pltpu.async_copy(
                    rows[b], agg.at[dbuf.at[par, b]], ssem[b], add=True))

            @pl.when(s + 1 < NSUP)
            def _():
                pltpu.make_async_copy(src_hbm.at[cid, sid, s + 1],
                                      sbuf.at[nxt], isem[0]).wait()
                pltpu.make_async_copy(dst_hbm.at[cid, sid, s + 1],
                                      dbuf.at[nxt], isem[1]).wait()
                for b in range(NBUF):
                    scat_a[b].wait()
                    pltpu.async_copy(h_hbm.at[sbuf.at[nxt, b]], rows[b],
                                     gsem[b])

            scat_b = []
            for b in range(NBUF):
                j = NBUF + b
                pltpu.make_async_copy(
                    h_hbm.at[sbuf.at[par, j]], rows[j], gsem[j]).wait()
                scat_b.append(pltpu.async_copy(
                    rows[j], agg.at[dbuf.at[par, j]], ssem[j], add=True))

            @pl.when(s + 1 < NSUP)
            def _():
                for b in range(NBUF):
                    j = NBUF + b
                    scat_b[b].wait()
                    pltpu.async_copy(h_hbm.at[sbuf.at[nxt, j]], rows[j],
                                     gsem[j])

            return carry

        lax.fori_loop(0, NSUP, super_step, 0)
        for j in range(2 * NBUF):
            pltpu.make_async_copy(
                rows[j], agg.at[dbuf.at[(NSUP - 1) % 2, j]], ssem[j]).wait()

        plsc.subcore_barrier()
        pltpu.sync_copy(agg.at[pl.ds(sid * RPT, RPT)],
                        out_hbm.at[cid, pl.ds(sid * RPT, RPT)])

    return agg_kernel(h, src3, dst3)


def _tc_mlp(h, p, eps, W1, b1, g, bt, W2, b2, relu_out):
    BR = 2000
    inv = float(1.0 / np.sqrt(1.0 + BN_EPS))

    def body(eps_ref, h_ref, p0_ref, p1_ref, w1_ref, b1_ref, g_ref, bt_ref,
             w2_ref, b2_ref, o_ref):
        scale = 1.0 + eps_ref[0, 0]
        a = h_ref[...] * scale + p0_ref[0] + p1_ref[0]
        gs = g_ref[...] * inv
        w1 = w1_ref[...] * gs
        c1 = b1_ref[...] * gs + bt_ref[...]
        t = jnp.dot(a, w1, preferred_element_type=jnp.float32,
                    precision=jax.lax.Precision.HIGHEST) + c1
        t = jnp.maximum(t, 0.0)
        o = jnp.dot(t, w2_ref[...], preferred_element_type=jnp.float32,
                    precision=jax.lax.Precision.HIGHEST) + b2_ref[...]
        if relu_out:
            o = jnp.maximum(o, 0.0)
        o_ref[...] = o

    row_spec = pl.BlockSpec((BR, D), lambda i: (i, 0))
    p0_spec = pl.BlockSpec((1, BR, D), lambda i: (0, i, 0))
    p1_spec = pl.BlockSpec((1, BR, D), lambda i: (1, i, 0))
    full = lambda shape: pl.BlockSpec(shape, lambda i: (0,) * len(shape))
    return pl.pallas_call(
        body,
        grid=(N // BR,),
        in_specs=[
            pl.BlockSpec(memory_space=pltpu.SMEM),
            row_spec, p0_spec, p1_spec,
            full((D, D)), full((1, D)), full((1, D)), full((1, D)),
            full((D, D)), full((1, D)),
        ],
        out_specs=row_spec,
        out_shape=jax.ShapeDtypeStruct((N, D), jnp.float32),
    )(eps.reshape(1, 1), h, p, p, W1, b1.reshape(1, D), g.reshape(1, D),
      bt.reshape(1, D), W2, b2.reshape(1, D))


def kernel(x, edge_index,
           W1_0, b1_0, g_0, bt_0, W2_0, b2_0, eps_0,
           W1_1, b1_1, g_1, bt_1, W2_1, b2_1, eps_1,
           W1_2, b1_2, g_2, bt_2, W2_2, b2_2, eps_2):
    src3 = edge_index[0].reshape(NC, NS, NSUP, 2 * NBUF, CH)
    dst3 = edge_index[1].reshape(NC, NS, NSUP, 2 * NBUF, CH)
    layers = [
        (W1_0, b1_0, g_0, bt_0, W2_0, b2_0, eps_0),
        (W1_1, b1_1, g_1, bt_1, W2_1, b2_1, eps_1),
        (W1_2, b1_2, g_2, bt_2, W2_2, b2_2, eps_2),
    ]
    h = x
    for i, (W1, b1, g, bt, W2, b2, eps) in enumerate(layers):
        p = _sc_aggregate(h, src3, dst3)
        h = _tc_mlp(h, p, eps, W1, b1, g, bt, W2, b2, relu_out=(i < 2))
    return h


# speedup vs baseline: 1.1362x; 1.0254x over previous
import functools

import jax
import jax.numpy as jnp
import numpy as np
from jax import lax
from jax.experimental import pallas as pl
from jax.experimental.pallas import tpu as pltpu
from jax.experimental.pallas import tpu_sc as plsc

N = 10000
D = 128
E = 320000
NC = 2
NS = 16
EW = E // (NC * NS)
CH = 64
NB = 2
NSUP = 40
EWP = NSUP * 2 * NB * CH
EP = EWP * NC * NS
NP = 10240
RPT = NP // NS
ZR = 16
BN_EPS = 1e-5


def _sc_aggregate(h, src3, dst3):
    mesh = plsc.VectorSubcoreMesh(core_axis_name="c", subcore_axis_name="s")

    @functools.partial(
        pl.kernel,
        out_type=jax.ShapeDtypeStruct((NC, NP, D), jnp.float32),
        mesh=mesh,
        scratch_types=[
            pltpu.VMEM((2, 2 * NB, CH), jnp.int32),
            pltpu.VMEM((2, 2 * NB, CH), jnp.int32),
            *[pltpu.VMEM((CH, D), jnp.float32) for _ in range(2 * NB)],
            pltpu.VMEM((ZR, D), jnp.float32),
            pltpu.VMEM_SHARED((NP, D), jnp.float32),
            *[pltpu.SemaphoreType.DMA for _ in range(4 * NB + 2)],
        ],
    )
    def agg_kernel(h_hbm, src_hbm, dst_hbm, out_hbm, sbuf, dbuf, *rest):
        rows = rest[:2 * NB]
        zbuf = rest[2 * NB]
        agg = rest[2 * NB + 1]
        gsem = rest[2 * NB + 2:2 * NB + 2 + 2 * NB]
        ssem = rest[2 * NB + 2 + 2 * NB:2 * NB + 2 + 4 * NB]
        isem = rest[2 * NB + 2 + 4 * NB:]
        cid = lax.axis_index("c")
        sid = lax.axis_index("s")

        zvec = jnp.zeros((16,), jnp.float32)

        def zfill(k, carry):
            zbuf[k // (D // 16), pl.ds((k % (D // 16)) * 16, 16)] = zvec
            return carry

        lax.fori_loop(0, ZR * (D // 16), zfill, 0)
        for r in range(RPT // ZR):
            pltpu.sync_copy(zbuf, agg.at[pl.ds(sid * RPT + r * ZR, ZR)])
        plsc.subcore_barrier()

        pltpu.sync_copy(src_hbm.at[cid, sid, 0], sbuf.at[0])
        pltpu.sync_copy(dst_hbm.at[cid, sid, 0], dbuf.at[0])
        for j in range(2 * NB):
            pltpu.async_copy(h_hbm.at[sbuf.at[0, j]], rows[j], gsem[j])

        def super_step(s, carry):
            par = s % 2
            nxt = (s + 1) % 2

            @pl.when(s + 1 < NSUP)
            def _():
                pltpu.async_copy(src_hbm.at[cid, sid, s + 1], sbuf.at[nxt],
                                 isem[0])
                pltpu.async_copy(dst_hbm.at[cid, sid, s + 1], dbuf.at[nxt],
                                 isem[1])

            scat_a = []
            for b in range(NB):
                pltpu.make_async_copy(
                    h_hbm.at[sbuf.at[par, b]], rows[b], gsem[b]).wait()
                scat_a.append(pltpu.async_copy(
                    rows[b], agg.at[dbuf.at[par, b]], ssem[b], add=True))

            @pl.when(s + 1 < NSUP)
            def _():
                pltpu.make_async_copy(src_hbm.at[cid, sid, s + 1],
                                      sbuf.at[nxt], isem[0]).wait()
                pltpu.make_async_copy(dst_hbm.at[cid, sid, s + 1],
                                      dbuf.at[nxt], isem[1]).wait()
                for b in range(NB):
                    scat_a[b].wait()
                    pltpu.async_copy(h_hbm.at[sbuf.at[nxt, b]], rows[b],
                                     gsem[b])

            scat_b = []
            for b in range(NB):
                j = NB + b
                pltpu.make_async_copy(
                    h_hbm.at[sbuf.at[par, j]], rows[j], gsem[j]).wait()
                scat_b.append(pltpu.async_copy(
                    rows[j], agg.at[dbuf.at[par, j]], ssem[j], add=True))

            @pl.when(s + 1 < NSUP)
            def _():
                for b in range(NB):
                    j = NB + b
                    scat_b[b].wait()
                    pltpu.async_copy(h_hbm.at[sbuf.at[nxt, j]], rows[j],
                                     gsem[j])

            return carry

        lax.fori_loop(0, NSUP, super_step, 0)
        for j in range(2 * NB):
            pltpu.make_async_copy(
                rows[j], agg.at[dbuf.at[(NSUP - 1) % 2, j]], ssem[j]).wait()

        plsc.subcore_barrier()
        pltpu.sync_copy(agg.at[pl.ds(sid * RPT, RPT)],
                        out_hbm.at[cid, pl.ds(sid * RPT, RPT)])

    return agg_kernel(h, src3, dst3)


def _tc_mlp(h, p, eps, W1, b1, g, bt, W2, b2, relu_out):
    BR = 2000
    inv = float(1.0 / np.sqrt(1.0 + BN_EPS))

    def body(eps_ref, h_ref, p0_ref, p1_ref, w1_ref, b1_ref, g_ref, bt_ref,
             w2_ref, b2_ref, o_ref):
        scale = 1.0 + eps_ref[0, 0]
        a = h_ref[...] * scale + p0_ref[0] + p1_ref[0]
        gs = g_ref[...] * inv
        w1 = w1_ref[...] * gs
        c1 = b1_ref[...] * gs + bt_ref[...]
        t = jnp.dot(a, w1, preferred_element_type=jnp.float32,
                    precision=jax.lax.Precision.HIGHEST) + c1
        t = jnp.maximum(t, 0.0)
        o = jnp.dot(t, w2_ref[...], preferred_element_type=jnp.float32,
                    precision=jax.lax.Precision.HIGHEST) + b2_ref[...]
        if relu_out:
            o = jnp.maximum(o, 0.0)
        o_ref[...] = o

    row_spec = pl.BlockSpec((BR, D), lambda i: (i, 0))
    p0_spec = pl.BlockSpec((1, BR, D), lambda i: (0, i, 0))
    p1_spec = pl.BlockSpec((1, BR, D), lambda i: (1, i, 0))
    full = lambda shape: pl.BlockSpec(shape, lambda i: (0,) * len(shape))
    return pl.pallas_call(
        body,
        grid=(N // BR,),
        in_specs=[
            pl.BlockSpec(memory_space=pltpu.SMEM),
            row_spec, p0_spec, p1_spec,
            full((D, D)), full((1, D)), full((1, D)), full((1, D)),
            full((D, D)), full((1, D)),
        ],
        out_specs=row_spec,
        out_shape=jax.ShapeDtypeStruct((N, D), jnp.float32),
    )(eps.reshape(1, 1), h, p, p, W1, b1.reshape(1, D), g.reshape(1, D),
      bt.reshape(1, D), W2, b2.reshape(1, D))


def kernel(x, edge_index,
           W1_0, b1_0, g_0, bt_0, W2_0, b2_0, eps_0,
           W1_1, b1_1, g_1, bt_1, W2_1, b2_1, eps_1,
           W1_2, b1_2, g_2, bt_2, W2_2, b2_2, eps_2):
    pad = EP - E
    pad_src = (jnp.arange(pad, dtype=jnp.int32) * 13) % N
    pad_dst = N + (jnp.arange(pad, dtype=jnp.int32) % (NP - N))
    src3 = jnp.concatenate([edge_index[0], pad_src]).reshape(
        NC, NS, NSUP, 2 * NB, CH)
    dst3 = jnp.concatenate([edge_index[1], pad_dst]).reshape(
        NC, NS, NSUP, 2 * NB, CH)
    layers = [
        (W1_0, b1_0, g_0, bt_0, W2_0, b2_0, eps_0),
        (W1_1, b1_1, g_1, bt_1, W2_1, b2_1, eps_1),
        (W1_2, b1_2, g_2, bt_2, W2_2, b2_2, eps_2),
    ]
    h = x
    for i, (W1, b1, g, bt, W2, b2, eps) in enumerate(layers):
        p = _sc_aggregate(h, src3, dst3)
        h = _tc_mlp(h, p, eps, W1, b1, g, bt, W2, b2, relu_out=(i < 2))
    return h


# speedup vs baseline: 1.1601x; 1.0210x over previous
import functools

import jax
import jax.numpy as jnp
import numpy as np
from jax import lax
from jax.experimental import pallas as pl
from jax.experimental.pallas import tpu as pltpu
from jax.experimental.pallas import tpu_sc as plsc

N = 10000
D = 128
E = 320000
NC = 2
NS = 16
EW = E // (NC * NS)
CH = 80
NB = 2
NSUP = 32
EWP = NSUP * 2 * NB * CH
EP = EWP * NC * NS
NP = 10240
RPT = NP // NS
ZR = 16
BN_EPS = 1e-5


def _sc_aggregate(h, src3, dst3):
    mesh = plsc.VectorSubcoreMesh(core_axis_name="c", subcore_axis_name="s")

    @functools.partial(
        pl.kernel,
        out_type=jax.ShapeDtypeStruct((NC, NP, D), jnp.float32),
        mesh=mesh,
        scratch_types=[
            pltpu.VMEM((2, 2 * NB, CH), jnp.int32),
            pltpu.VMEM((2, 2 * NB, CH), jnp.int32),
            *[pltpu.VMEM((CH, D), jnp.float32) for _ in range(2 * NB)],
            pltpu.VMEM((ZR, D), jnp.float32),
            pltpu.VMEM_SHARED((NP, D), jnp.float32),
            *[pltpu.SemaphoreType.DMA for _ in range(4 * NB + 2)],
        ],
    )
    def agg_kernel(h_hbm, src_hbm, dst_hbm, out_hbm, sbuf, dbuf, *rest):
        rows = rest[:2 * NB]
        zbuf = rest[2 * NB]
        agg = rest[2 * NB + 1]
        gsem = rest[2 * NB + 2:2 * NB + 2 + 2 * NB]
        ssem = rest[2 * NB + 2 + 2 * NB:2 * NB + 2 + 4 * NB]
        isem = rest[2 * NB + 2 + 4 * NB:]
        cid = lax.axis_index("c")
        sid = lax.axis_index("s")

        pltpu.sync_copy(src_hbm.at[cid, sid, 0], sbuf.at[0])
        pltpu.sync_copy(dst_hbm.at[cid, sid, 0], dbuf.at[0])
        for j in range(2 * NB):
            pltpu.async_copy(h_hbm.at[sbuf.at[0, j]], rows[j], gsem[j])

        zvec = jnp.zeros((16,), jnp.float32)

        def zfill(k, carry):
            zbuf[k // (D // 16), pl.ds((k % (D // 16)) * 16, 16)] = zvec
            return carry

        lax.fori_loop(0, ZR * (D // 16), zfill, 0)
        for r in range(RPT // ZR):
            pltpu.sync_copy(zbuf, agg.at[pl.ds(sid * RPT + r * ZR, ZR)])
        plsc.subcore_barrier()

        def super_step(s, carry):
            par = s % 2
            nxt = (s + 1) % 2

            @pl.when(s + 1 < NSUP)
            def _():
                pltpu.async_copy(src_hbm.at[cid, sid, s + 1], sbuf.at[nxt],
                                 isem[0])
                pltpu.async_copy(dst_hbm.at[cid, sid, s + 1], dbuf.at[nxt],
                                 isem[1])

            scat_a = []
            for b in range(NB):
                pltpu.make_async_copy(
                    h_hbm.at[sbuf.at[par, b]], rows[b], gsem[b]).wait()
                scat_a.append(pltpu.async_copy(
                    rows[b], agg.at[dbuf.at[par, b]], ssem[b], add=True))

            @pl.when(s + 1 < NSUP)
            def _():
                pltpu.make_async_copy(src_hbm.at[cid, sid, s + 1],
                                      sbuf.at[nxt], isem[0]).wait()
                pltpu.make_async_copy(dst_hbm.at[cid, sid, s + 1],
                                      dbuf.at[nxt], isem[1]).wait()
                for b in range(NB):
                    scat_a[b].wait()
                    pltpu.async_copy(h_hbm.at[sbuf.at[nxt, b]], rows[b],
                                     gsem[b])

            scat_b = []
            for b in range(NB):
                j = NB + b
                pltpu.make_async_copy(
                    h_hbm.at[sbuf.at[par, j]], rows[j], gsem[j]).wait()
                scat_b.append(pltpu.async_copy(
                    rows[j], agg.at[dbuf.at[par, j]], ssem[j], add=True))

            @pl.when(s + 1 < NSUP)
            def _():
                for b in range(NB):
                    j = NB + b
                    scat_b[b].wait()
                    pltpu.async_copy(h_hbm.at[sbuf.at[nxt, j]], rows[j],
                                     gsem[j])

            return carry

        lax.fori_loop(0, NSUP, super_step, 0)
        for j in range(2 * NB):
            pltpu.make_async_copy(
                rows[j], agg.at[dbuf.at[(NSUP - 1) % 2, j]], ssem[j]).wait()

        plsc.subcore_barrier()
        pltpu.sync_copy(agg.at[pl.ds(sid * RPT, RPT)],
                        out_hbm.at[cid, pl.ds(sid * RPT, RPT)])

    return agg_kernel(h, src3, dst3)


def _tc_mlp(h, p, eps, W1, b1, g, bt, W2, b2, relu_out):
    BR = 2000
    inv = float(1.0 / np.sqrt(1.0 + BN_EPS))

    def body(eps_ref, h_ref, p0_ref, p1_ref, w1_ref, b1_ref, g_ref, bt_ref,
             w2_ref, b2_ref, o_ref):
        scale = 1.0 + eps_ref[0, 0]
        a = h_ref[...] * scale + p0_ref[0] + p1_ref[0]
        gs = g_ref[...] * inv
        w1 = w1_ref[...] * gs
        c1 = b1_ref[...] * gs + bt_ref[...]
        t = jnp.dot(a, w1, preferred_element_type=jnp.float32,
                    precision=jax.lax.Precision.HIGHEST) + c1
        t = jnp.maximum(t, 0.0)
        o = jnp.dot(t, w2_ref[...], preferred_element_type=jnp.float32,
                    precision=jax.lax.Precision.HIGHEST) + b2_ref[...]
        if relu_out:
            o = jnp.maximum(o, 0.0)
        o_ref[...] = o

    row_spec = pl.BlockSpec((BR, D), lambda i: (i, 0))
    p0_spec = pl.BlockSpec((1, BR, D), lambda i: (0, i, 0))
    p1_spec = pl.BlockSpec((1, BR, D), lambda i: (1, i, 0))
    full = lambda shape: pl.BlockSpec(shape, lambda i: (0,) * len(shape))
    return pl.pallas_call(
        body,
        grid=(N // BR,),
        in_specs=[
            pl.BlockSpec(memory_space=pltpu.SMEM),
            row_spec, p0_spec, p1_spec,
            full((D, D)), full((1, D)), full((1, D)), full((1, D)),
            full((D, D)), full((1, D)),
        ],
        out_specs=row_spec,
        out_shape=jax.ShapeDtypeStruct((N, D), jnp.float32),
    )(eps.reshape(1, 1), h, p, p, W1, b1.reshape(1, D), g.reshape(1, D),
      bt.reshape(1, D), W2, b2.reshape(1, D))


def kernel(x, edge_index,
           W1_0, b1_0, g_0, bt_0, W2_0, b2_0, eps_0,
           W1_1, b1_1, g_1, bt_1, W2_1, b2_1, eps_1,
           W1_2, b1_2, g_2, bt_2, W2_2, b2_2, eps_2):
    pad = EP - E
    pad_src = (jnp.arange(pad, dtype=jnp.int32) * 13) % N
    pad_dst = N + (jnp.arange(pad, dtype=jnp.int32) % (NP - N))
    src3 = jnp.concatenate([edge_index[0], pad_src]).reshape(
        NC, NS, NSUP, 2 * NB, CH)
    dst3 = jnp.concatenate([edge_index[1], pad_dst]).reshape(
        NC, NS, NSUP, 2 * NB, CH)
    layers = [
        (W1_0, b1_0, g_0, bt_0, W2_0, b2_0, eps_0),
        (W1_1, b1_1, g_1, bt_1, W2_1, b2_1, eps_1),
        (W1_2, b1_2, g_2, bt_2, W2_2, b2_2, eps_2),
    ]
    h = x
    for i, (W1, b1, g, bt, W2, b2, eps) in enumerate(layers):
        p = _sc_aggregate(h, src3, dst3)
        h = _tc_mlp(h, p, eps, W1, b1, g, bt, W2, b2, relu_out=(i < 2))
    return h
